# Initial kernel scaffold; baseline (speedup 1.0000x reference)
import jax
import jax.numpy as jnp
from jax.experimental import pallas as pl


def kernel(x, xyz_moving, edge_index, W1, att1, We1, b1, W2, att2, We2, b2):
    raise NotImplementedError("write your pallas kernel here")


# speedup vs baseline: 6.4643x; 6.4643x over previous
import functools

import jax
import jax.numpy as jnp
from jax import lax
from jax.experimental import pallas as pl
from jax.experimental.pallas import tpu as pltpu
from jax.experimental.pallas import tpu_sc as plsc

NN = 40962
EE = 245760
H = 4
C = 32
D = 128
NP = 41472
NC = 2
NS = 16
K = 128
EW = EE // (NC * NS)
EH = EE // NS
RQ = NP // NS
RB = RQ // 3


def _norm_body(v_ref, o_ref):
    v = v_ref[...]
    n2 = jnp.sum(v * v, axis=1, keepdims=True)
    xn = v / jnp.sqrt(n2)
    o_ref[...] = jnp.concatenate(
        [xn, jnp.zeros((v.shape[0], 12), jnp.float32)], axis=1)


def _normalize_xyz(xyzp):
    return pl.pallas_call(
        _norm_body,
        grid=(NP // 512,),
        in_specs=[pl.BlockSpec((512, 4), lambda i: (i, 0))],
        out_specs=pl.BlockSpec((512, 16), lambda i: (i, 0)),
        out_shape=jax.ShapeDtypeStruct((NP, 16), jnp.float32),
    )(xyzp)


def _mm_body(x_ref, w_ref, o_ref):
    o_ref[0] = jnp.dot(x_ref[...], w_ref[0],
                       preferred_element_type=jnp.float32)


def _matmul_hm(xp, w_hm):
    return pl.pallas_call(
        _mm_body,
        grid=(NP // 512, H),
        in_specs=[pl.BlockSpec((512, D), lambda i, h: (i, 0)),
                  pl.BlockSpec((1, D, C), lambda i, h: (h, 0, 0))],
        out_specs=pl.BlockSpec((1, 512, C), lambda i, h: (h, i, 0)),
        out_shape=jax.ShapeDtypeStruct((H, NP, C), jnp.float32),
    )(xp, w_hm)


def _ef_body(d_ref, w1_ref, w2_ref, o1_ref, o2_ref):
    v = d_ref[:, 0:3]
    outs = [v]
    for f in (1.0, 2.0, 4.0, 8.0):
        outs.append(jnp.sin(f * v))
        outs.append(jnp.cos(f * v))
    ea = jnp.concatenate(outs, axis=1)
    o1_ref[0] = jnp.dot(ea, w1_ref[0], preferred_element_type=jnp.float32)
    o2_ref[0] = jnp.dot(ea, w2_ref[0], preferred_element_type=jnp.float32)


def _edge_features(dif, we1_hm, we2_hm):
    return pl.pallas_call(
        _ef_body,
        grid=(EE // 1024, H),
        in_specs=[pl.BlockSpec((1024, 16), lambda i, h: (i, 0)),
                  pl.BlockSpec((1, 27, C), lambda i, h: (h, 0, 0)),
                  pl.BlockSpec((1, 27, C), lambda i, h: (h, 0, 0))],
        out_specs=[pl.BlockSpec((1, 1024, C), lambda i, h: (h, i, 0)),
                   pl.BlockSpec((1, 1024, C), lambda i, h: (h, i, 0))],
        out_shape=[jax.ShapeDtypeStruct((H, EE, C), jnp.float32),
                   jax.ShapeDtypeStruct((H, EE, C), jnp.float32)],
    )(dif, we1_hm, we2_hm)


def _mid_body(n_ref, d_ref, b_ref, o_ref):
    y = n_ref[0] / (d_ref[0] + 1e-16) + b_ref[0]
    o_ref[0] = jnp.where(y >= 0, y, 0.01 * y)


def _mid(num, den, b_hm):
    return pl.pallas_call(
        _mid_body,
        grid=(NP // 512, H),
        in_specs=[pl.BlockSpec((1, 512, C), lambda i, h: (h, i, 0)),
                  pl.BlockSpec((1, 512, 1), lambda i, h: (h, i, 0)),
                  pl.BlockSpec((1, 1, C), lambda i, h: (h, 0, 0))],
        out_specs=pl.BlockSpec((1, 512, C), lambda i, h: (h, i, 0)),
        out_shape=jax.ShapeDtypeStruct((H, NP, C), jnp.float32),
    )(num, den, b_hm)


def _final_body(n_ref, d_ref, b_ref, x_ref, o_ref):
    o_ref[0] = n_ref[0] / (d_ref[0] + 1e-16) + b_ref[0] + x_ref[0]


def _final(num, den, b_hm, xp_hm):
    return pl.pallas_call(
        _final_body,
        grid=(NP // 512, H),
        in_specs=[pl.BlockSpec((1, 512, C), lambda i, h: (h, i, 0)),
                  pl.BlockSpec((1, 512, 1), lambda i, h: (h, i, 0)),
                  pl.BlockSpec((1, 1, C), lambda i, h: (h, 0, 0)),
                  pl.BlockSpec((1, 512, C), lambda i, h: (h, i, 0))],
        out_specs=pl.BlockSpec((1, 512, C), lambda i, h: (h, i, 0)),
        out_shape=jax.ShapeDtypeStruct((H, NP, C), jnp.float32),
    )(num, den, b_hm, xp_hm)


_MESH = plsc.VectorSubcoreMesh(core_axis_name="c", subcore_axis_name="s")
_SC_PARAMS = pltpu.CompilerParams(use_tc_tiling_on_sc=False,
                                  needs_layout_passes=False)


@functools.partial(
    pl.kernel, mesh=_MESH,
    compiler_params=_SC_PARAMS,
    out_type=jax.ShapeDtypeStruct((EE, 16), jnp.float32),
    scratch_types=[pltpu.VMEM((K,), jnp.int32),
                   pltpu.VMEM((K,), jnp.int32),
                   pltpu.VMEM((K, 16), jnp.float32),
                   pltpu.VMEM((K, 16), jnp.float32),
                   pltpu.SemaphoreType.DMA,
                   pltpu.SemaphoreType.DMA])
def _edge_vectors(xyzn_hbm, src_hbm, dst_hbm, dif_hbm, sv, dv, xs, xd,
                  sem1, sem2):
    wid = lax.axis_index("s") * NC + lax.axis_index("c")
    base = wid * EW

    @pl.loop(0, EW // K)
    def _chunk(ci):
        e0 = base + ci * K
        pltpu.sync_copy(src_hbm.at[pl.ds(e0, K)], sv)
        pltpu.sync_copy(dst_hbm.at[pl.ds(e0, K)], dv)
        g1 = pltpu.async_copy(xyzn_hbm.at[sv], xs, sem1)
        g2 = pltpu.async_copy(xyzn_hbm.at[dv], xd, sem2)
        g1.wait()
        g2.wait()

        @pl.loop(0, K)
        def _row(i):
            xs[i, pl.ds(0, 16)] = xs[i, pl.ds(0, 16)] - xd[i, pl.ds(0, 16)]

        pltpu.sync_copy(xs, dif_hbm.at[pl.ds(e0, K)])


def _gat_sc(xl2d, ef2d, srcg, dstg, dst, att):
    @functools.partial(
        pl.kernel, mesh=_MESH,
        compiler_params=_SC_PARAMS,
        out_type=(jax.ShapeDtypeStruct((H * NP, C), jnp.float32),
                  jax.ShapeDtypeStruct((H * NP,), jnp.float32)),
        scratch_types=[pltpu.VMEM_SHARED((NP, C), jnp.float32),
                       pltpu.VMEM_SHARED((NP,), jnp.float32),
                       pltpu.VMEM((K,), jnp.int32),
                       pltpu.VMEM((K,), jnp.int32),
                       pltpu.VMEM((K,), jnp.int32),
                       pltpu.VMEM((K, C), jnp.float32),
                       pltpu.VMEM((K, C), jnp.float32),
                       pltpu.VMEM((K, C), jnp.float32),
                       pltpu.VMEM((K,), jnp.float32),
                       pltpu.VMEM((C,), jnp.float32),
                       pltpu.VMEM((RB, C), jnp.float32),
                       pltpu.VMEM((RB,), jnp.float32),
                       pltpu.SemaphoreType.DMA,
                       pltpu.SemaphoreType.DMA])
    def body(xl_hbm, ef_hbm, srcg_hbm, dstg_hbm, dst_hbm, att_hbm,
             num_hbm, den_hbm,
             num_acc, den_acc, sgv, dgv, dpv, xiv, xjv, efv, exb,
             attv, cbuf, dbuf, sem1, sem2):
        cc = lax.axis_index("c")
        ss = lax.axis_index("s")
        r0 = ss * RQ
        for hp in range(2):
            hh = cc * 2 + hp

            @pl.loop(0, RB)
            def _zrow(r):
                z = jnp.zeros((16,), jnp.float32)
                cbuf[r, pl.ds(0, 16)] = z
                cbuf[r, pl.ds(16, 16)] = z

            @pl.loop(0, RB // 16)
            def _zden(i):
                dbuf[pl.ds(i * 16, 16)] = jnp.zeros((16,), jnp.float32)

            for q in range(3):
                pltpu.sync_copy(cbuf, num_acc.at[pl.ds(r0 + q * RB, RB)])
                pltpu.sync_copy(dbuf, den_acc.at[pl.ds(r0 + q * RB, RB)])
            plsc.subcore_barrier()

            pltpu.sync_copy(att_hbm.at[pl.ds(hh * C, C)], attv)
            av0 = attv[pl.ds(0, 16)]
            av1 = attv[pl.ds(16, 16)]
            atts = [av0[cidx] for cidx in range(16)] \
                + [av1[cidx] for cidx in range(16)]
            lane = lax.iota(jnp.int32, 16)
            ebase = ss * EH

            @pl.loop(0, EH // K)
            def _chunk(ci):
                e0 = ebase + ci * K
                ge0 = hh * EE + e0
                pltpu.sync_copy(srcg_hbm.at[pl.ds(ge0, K)], sgv)
                pltpu.sync_copy(dstg_hbm.at[pl.ds(ge0, K)], dgv)
                pltpu.sync_copy(dst_hbm.at[pl.ds(e0, K)], dpv)
                g1 = pltpu.async_copy(xl_hbm.at[sgv], xjv, sem1)
                g2 = pltpu.async_copy(xl_hbm.at[dgv], xiv, sem2)
                pltpu.sync_copy(ef_hbm.at[pl.ds(ge0, K)], efv)
                g1.wait()
                g2.wait()

                @pl.loop(0, K)
                def _leaky(i):
                    t0 = xiv[i, pl.ds(0, 16)] + xjv[i, pl.ds(0, 16)] \
                        + efv[i, pl.ds(0, 16)]
                    t1 = xiv[i, pl.ds(16, 16)] + xjv[i, pl.ds(16, 16)] \
                        + efv[i, pl.ds(16, 16)]
                    xiv[i, pl.ds(0, 16)] = jnp.where(t0 >= 0, t0, 0.2 * t0)
                    xiv[i, pl.ds(16, 16)] = jnp.where(t1 >= 0, t1, 0.2 * t1)

                @pl.loop(0, K // 16)
                def _alpha(g):
                    e = g * 16 + lane
                    acc = jnp.zeros((16,), jnp.float32)
                    for cidx in range(C):
                        col = jnp.full((16,), cidx, jnp.int32)
                        acc = acc + plsc.load_gather(xiv, [e, col]) * atts[cidx]
                    exb[pl.ds(g * 16, 16)] = jnp.exp(acc)

                @pl.loop(0, K // 16)
                def _scale(g):
                    exv = exb[pl.ds(g * 16, 16)]
                    for l in range(16):
                        i = g * 16 + l
                        ex = exv[l]
                        xjv[i, pl.ds(0, 16)] = xjv[i, pl.ds(0, 16)] * ex
                        xjv[i, pl.ds(16, 16)] = xjv[i, pl.ds(16, 16)] * ex

                pltpu.sync_copy(xjv, num_acc.at[dpv], add=True)
                pltpu.sync_copy(exb, den_acc.at[dpv], add=True)

            plsc.subcore_barrier()
            for q in range(3):
                pltpu.sync_copy(num_acc.at[pl.ds(r0 + q * RB, RB)], cbuf)
                pltpu.sync_copy(
                    cbuf, num_hbm.at[pl.ds(hh * NP + r0 + q * RB, RB)])
                pltpu.sync_copy(den_acc.at[pl.ds(r0 + q * RB, RB)], dbuf)
                pltpu.sync_copy(
                    dbuf, den_hbm.at[pl.ds(hh * NP + r0 + q * RB, RB)])
            plsc.subcore_barrier()

    return body(xl2d, ef2d, srcg, dstg, dst, att)


def kernel(x, xyz_moving, edge_index, W1, att1, We1, b1, W2, att2, We2, b2):
    src = edge_index[0]
    dst = edge_index[1]
    xp = jnp.pad(x, ((0, NP - NN), (0, 0)))
    xyzp = jnp.pad(xyz_moving, ((0, NP - NN), (0, 1)))

    w1_hm = W1.reshape(D, H, C).transpose(1, 0, 2)
    w2_hm = W2.reshape(D, H, C).transpose(1, 0, 2)
    we1_hm = We1.reshape(27, H, C).transpose(1, 0, 2)
    we2_hm = We2.reshape(27, H, C).transpose(1, 0, 2)
    b1_hm = b1.reshape(H, 1, C)
    b2_hm = b2.reshape(H, 1, C)

    offs = (jnp.arange(H, dtype=jnp.int32) * NP)[:, None]
    srcg = (src[None, :] + offs).reshape(-1)
    dstg = (dst[None, :] + offs).reshape(-1)

    xyzn = _normalize_xyz(xyzp)
    dif = _edge_vectors(xyzn, src, dst)
    ef1, ef2 = _edge_features(dif, we1_hm, we2_hm)

    xl1 = _matmul_hm(xp, w1_hm).reshape(H * NP, C)
    num1, den1 = _gat_sc(xl1, ef1.reshape(H * EE, C), srcg, dstg, dst,
                         att1.reshape(-1))
    h1_hm = _mid(num1.reshape(H, NP, C), den1.reshape(H, NP, 1), b1_hm)
    h1 = h1_hm.transpose(1, 0, 2).reshape(NP, D)

    xl2 = _matmul_hm(h1, w2_hm).reshape(H * NP, C)
    num2, den2 = _gat_sc(xl2, ef2.reshape(H * EE, C), srcg, dstg, dst,
                         att2.reshape(-1))
    xp_hm = xp.reshape(NP, H, C).transpose(1, 0, 2)
    out_hm = _final(num2.reshape(H, NP, C), den2.reshape(H, NP, 1), b2_hm,
                    xp_hm)
    return out_hm.transpose(1, 0, 2).reshape(NP, D)[:NN]


# speedup vs baseline: 13.7389x; 2.1253x over previous
import functools

import jax
import jax.numpy as jnp
import numpy as np
from jax import lax
from jax.experimental import pallas as pl
from jax.experimental.pallas import tpu as pltpu
from jax.experimental.pallas import tpu_sc as plsc

NN = 40962
EE = 245760
H = 4
C = 32
D = 128
NP = 41472
NC = 2
NS = 16
K = 128
EW = EE // (NC * NS)
EH = EE // NS
RQ = NP // NS
RB = RQ // 3

_S16 = np.zeros((16, 12), np.float32)
for _k, _f in enumerate((1.0, 2.0, 4.0, 8.0)):
    for _c in range(3):
        _S16[_c, 3 * _k + _c] = _f


def _split_we(we):
    wv = jnp.pad(we[0:3], ((0, 13), (0, 0)))
    ws = jnp.concatenate([we[3:6], we[9:12], we[15:18], we[21:24]], axis=0)
    wc = jnp.concatenate([we[6:9], we[12:15], we[18:21], we[24:27]], axis=0)
    hm = lambda w, r: w.reshape(r, H, C).transpose(1, 0, 2)
    return hm(wv, 16), hm(ws, 12), hm(wc, 12)


def _norm_body(v_ref, o_ref):
    v = v_ref[...]
    n2 = jnp.sum(v * v, axis=1, keepdims=True)
    xn = v / jnp.sqrt(n2)
    o_ref[...] = jnp.concatenate(
        [xn, jnp.zeros((v.shape[0], 12), jnp.float32)], axis=1)


def _normalize_xyz(xyzp):
    return pl.pallas_call(
        _norm_body,
        grid=(NP // 512,),
        in_specs=[pl.BlockSpec((512, 4), lambda i: (i, 0))],
        out_specs=pl.BlockSpec((512, 16), lambda i: (i, 0)),
        out_shape=jax.ShapeDtypeStruct((NP, 16), jnp.float32),
    )(xyzp)


def _mm_body(x_ref, w_ref, o_ref):
    o_ref[0] = jnp.dot(x_ref[...], w_ref[0],
                       preferred_element_type=jnp.float32)


def _matmul_hm(xp, w_hm):
    return pl.pallas_call(
        _mm_body,
        grid=(NP // 512, H),
        in_specs=[pl.BlockSpec((512, D), lambda i, h: (i, 0)),
                  pl.BlockSpec((1, D, C), lambda i, h: (h, 0, 0))],
        out_specs=pl.BlockSpec((1, 512, C), lambda i, h: (h, i, 0)),
        out_shape=jax.ShapeDtypeStruct((H, NP, C), jnp.float32),
    )(xp, w_hm)


def _ef_body(d_ref, s_ref, wv1_ref, ws1_ref, wc1_ref, wv2_ref, ws2_ref,
             wc2_ref, o1_ref, o2_ref):
    d = d_ref[...]
    m = jnp.dot(d, s_ref[...], preferred_element_type=jnp.float32)
    sm = jnp.sin(m)
    cm = jnp.cos(m)
    for h in range(H):
        o1_ref[h] = (
            jnp.dot(d, wv1_ref[h], preferred_element_type=jnp.float32)
            + jnp.dot(sm, ws1_ref[h], preferred_element_type=jnp.float32)
            + jnp.dot(cm, wc1_ref[h], preferred_element_type=jnp.float32))
        o2_ref[h] = (
            jnp.dot(d, wv2_ref[h], preferred_element_type=jnp.float32)
            + jnp.dot(sm, ws2_ref[h], preferred_element_type=jnp.float32)
            + jnp.dot(cm, wc2_ref[h], preferred_element_type=jnp.float32))


def _edge_features(dif, s16, wsplit1, wsplit2):
    wv1, ws1, wc1 = wsplit1
    wv2, ws2, wc2 = wsplit2
    full = lambda shape: pl.BlockSpec(shape, lambda i: tuple(
        0 for _ in shape))
    return pl.pallas_call(
        _ef_body,
        grid=(EE // 1024,),
        in_specs=[pl.BlockSpec((1024, 16), lambda i: (i, 0)),
                  full((16, 12)),
                  full((H, 16, C)), full((H, 12, C)), full((H, 12, C)),
                  full((H, 16, C)), full((H, 12, C)), full((H, 12, C))],
        out_specs=[pl.BlockSpec((H, 1024, C), lambda i: (0, i, 0)),
                   pl.BlockSpec((H, 1024, C), lambda i: (0, i, 0))],
        out_shape=[jax.ShapeDtypeStruct((H, EE, C), jnp.float32),
                   jax.ShapeDtypeStruct((H, EE, C), jnp.float32)],
    )(dif, s16, wv1, ws1, wc1, wv2, ws2, wc2)


def _mid_body(n_ref, d_ref, b_ref, o_ref):
    y = n_ref[0] / (d_ref[0] + 1e-16) + b_ref[0]
    o_ref[0] = jnp.where(y >= 0, y, 0.01 * y)


def _mid(num, den, b_hm):
    return pl.pallas_call(
        _mid_body,
        grid=(NP // 512, H),
        in_specs=[pl.BlockSpec((1, 512, C), lambda i, h: (h, i, 0)),
                  pl.BlockSpec((1, 512, 1), lambda i, h: (h, i, 0)),
                  pl.BlockSpec((1, 1, C), lambda i, h: (h, 0, 0))],
        out_specs=pl.BlockSpec((1, 512, C), lambda i, h: (h, i, 0)),
        out_shape=jax.ShapeDtypeStruct((H, NP, C), jnp.float32),
    )(num, den, b_hm)


def _final_body(n_ref, d_ref, b_ref, x_ref, o_ref):
    o_ref[0] = n_ref[0] / (d_ref[0] + 1e-16) + b_ref[0] + x_ref[0]


def _final(num, den, b_hm, xp_hm):
    return pl.pallas_call(
        _final_body,
        grid=(NP // 512, H),
        in_specs=[pl.BlockSpec((1, 512, C), lambda i, h: (h, i, 0)),
                  pl.BlockSpec((1, 512, 1), lambda i, h: (h, i, 0)),
                  pl.BlockSpec((1, 1, C), lambda i, h: (h, 0, 0)),
                  pl.BlockSpec((1, 512, C), lambda i, h: (h, i, 0))],
        out_specs=pl.BlockSpec((1, 512, C), lambda i, h: (h, i, 0)),
        out_shape=jax.ShapeDtypeStruct((H, NP, C), jnp.float32),
    )(num, den, b_hm, xp_hm)


_MESH = plsc.VectorSubcoreMesh(core_axis_name="c", subcore_axis_name="s")
_SC_PARAMS = pltpu.CompilerParams(use_tc_tiling_on_sc=False,
                                  needs_layout_passes=False)


@functools.partial(
    pl.kernel, mesh=_MESH,
    compiler_params=_SC_PARAMS,
    out_type=jax.ShapeDtypeStruct((EE, 16), jnp.float32),
    scratch_types=[pltpu.VMEM((K,), jnp.int32),
                   pltpu.VMEM((K,), jnp.int32),
                   pltpu.VMEM((K, 16), jnp.float32),
                   pltpu.VMEM((K, 16), jnp.float32),
                   pltpu.SemaphoreType.DMA,
                   pltpu.SemaphoreType.DMA])
def _edge_vectors(xyzn_hbm, src_hbm, dst_hbm, dif_hbm, sv, dv, xs, xd,
                  sem1, sem2):
    wid = lax.axis_index("s") * NC + lax.axis_index("c")
    base = wid * EW

    @pl.loop(0, EW // K)
    def _chunk(ci):
        e0 = base + ci * K
        pltpu.sync_copy(src_hbm.at[pl.ds(e0, K)], sv)
        pltpu.sync_copy(dst_hbm.at[pl.ds(e0, K)], dv)
        g1 = pltpu.async_copy(xyzn_hbm.at[sv], xs, sem1)
        g2 = pltpu.async_copy(xyzn_hbm.at[dv], xd, sem2)
        g1.wait()
        g2.wait()

        @pl.loop(0, K)
        def _row(i):
            xs[i, pl.ds(0, 16)] = xs[i, pl.ds(0, 16)] - xd[i, pl.ds(0, 16)]

        pltpu.sync_copy(xs, dif_hbm.at[pl.ds(e0, K)])


def _gat_sc(xl2d, ef2d, srcg, dstg, dst, att):
    @functools.partial(
        pl.kernel, mesh=_MESH,
        compiler_params=_SC_PARAMS,
        out_type=(jax.ShapeDtypeStruct((H * NP, C), jnp.float32),
                  jax.ShapeDtypeStruct((H * NP,), jnp.float32)),
        scratch_types=[pltpu.VMEM_SHARED((NP, C), jnp.float32),
                       pltpu.VMEM_SHARED((NP,), jnp.float32),
                       pltpu.VMEM((K,), jnp.int32),
                       pltpu.VMEM((K,), jnp.int32),
                       pltpu.VMEM((K,), jnp.int32),
                       pltpu.VMEM((K, C), jnp.float32),
                       pltpu.VMEM((K, C), jnp.float32),
                       pltpu.VMEM((K, C), jnp.float32),
                       pltpu.VMEM((K,), jnp.float32),
                       pltpu.VMEM((C,), jnp.float32),
                       pltpu.VMEM((RB, C), jnp.float32),
                       pltpu.VMEM((RB,), jnp.float32),
                       pltpu.SemaphoreType.DMA,
                       pltpu.SemaphoreType.DMA])
    def body(xl_hbm, ef_hbm, srcg_hbm, dstg_hbm, dst_hbm, att_hbm,
             num_hbm, den_hbm,
             num_acc, den_acc, sgv, dgv, dpv, xiv, xjv, efv, exb,
             attv, cbuf, dbuf, sem1, sem2):
        cc = lax.axis_index("c")
        ss = lax.axis_index("s")
        r0 = ss * RQ
        for hp in range(2):
            hh = cc * 2 + hp

            @pl.loop(0, RB)
            def _zrow(r):
                z = jnp.zeros((16,), jnp.float32)
                cbuf[r, pl.ds(0, 16)] = z
                cbuf[r, pl.ds(16, 16)] = z

            @pl.loop(0, RB // 16)
            def _zden(i):
                dbuf[pl.ds(i * 16, 16)] = jnp.zeros((16,), jnp.float32)

            for q in range(3):
                pltpu.sync_copy(cbuf, num_acc.at[pl.ds(r0 + q * RB, RB)])
                pltpu.sync_copy(dbuf, den_acc.at[pl.ds(r0 + q * RB, RB)])
            plsc.subcore_barrier()

            pltpu.sync_copy(att_hbm.at[pl.ds(hh * C, C)], attv)
            av0 = attv[pl.ds(0, 16)]
            av1 = attv[pl.ds(16, 16)]
            atts = [av0[cidx] for cidx in range(16)] \
                + [av1[cidx] for cidx in range(16)]
            lane = lax.iota(jnp.int32, 16)
            ebase = ss * EH

            @pl.loop(0, EH // K)
            def _chunk(ci):
                e0 = ebase + ci * K
                ge0 = hh * EE + e0
                pltpu.sync_copy(srcg_hbm.at[pl.ds(ge0, K)], sgv)
                pltpu.sync_copy(dstg_hbm.at[pl.ds(ge0, K)], dgv)
                pltpu.sync_copy(dst_hbm.at[pl.ds(e0, K)], dpv)
                g1 = pltpu.async_copy(xl_hbm.at[sgv], xjv, sem1)
                g2 = pltpu.async_copy(xl_hbm.at[dgv], xiv, sem2)
                pltpu.sync_copy(ef_hbm.at[pl.ds(ge0, K)], efv)
                g1.wait()
                g2.wait()

                @pl.loop(0, K)
                def _leaky(i):
                    t0 = xiv[i, pl.ds(0, 16)] + xjv[i, pl.ds(0, 16)] \
                        + efv[i, pl.ds(0, 16)]
                    t1 = xiv[i, pl.ds(16, 16)] + xjv[i, pl.ds(16, 16)] \
                        + efv[i, pl.ds(16, 16)]
                    xiv[i, pl.ds(0, 16)] = jnp.where(t0 >= 0, t0, 0.2 * t0)
                    xiv[i, pl.ds(16, 16)] = jnp.where(t1 >= 0, t1, 0.2 * t1)

                @pl.loop(0, K // 16)
                def _alpha(g):
                    e = g * 16 + lane
                    acc = jnp.zeros((16,), jnp.float32)
                    for cidx in range(C):
                        col = jnp.full((16,), cidx, jnp.int32)
                        acc = acc + plsc.load_gather(xiv, [e, col]) * atts[cidx]
                    exb[pl.ds(g * 16, 16)] = jnp.exp(acc)

                @pl.loop(0, K // 16)
                def _scale(g):
                    exv = exb[pl.ds(g * 16, 16)]
                    for l in range(16):
                        i = g * 16 + l
                        ex = exv[l]
                        xjv[i, pl.ds(0, 16)] = xjv[i, pl.ds(0, 16)] * ex
                        xjv[i, pl.ds(16, 16)] = xjv[i, pl.ds(16, 16)] * ex

                pltpu.sync_copy(xjv, num_acc.at[dpv], add=True)
                pltpu.sync_copy(exb, den_acc.at[dpv], add=True)

            plsc.subcore_barrier()
            for q in range(3):
                pltpu.sync_copy(num_acc.at[pl.ds(r0 + q * RB, RB)], cbuf)
                pltpu.sync_copy(
                    cbuf, num_hbm.at[pl.ds(hh * NP + r0 + q * RB, RB)])
                pltpu.sync_copy(den_acc.at[pl.ds(r0 + q * RB, RB)], dbuf)
                pltpu.sync_copy(
                    dbuf, den_hbm.at[pl.ds(hh * NP + r0 + q * RB, RB)])
            plsc.subcore_barrier()

    return body(xl2d, ef2d, srcg, dstg, dst, att)


def kernel(x, xyz_moving, edge_index, W1, att1, We1, b1, W2, att2, We2, b2):
    src = edge_index[0]
    dst = edge_index[1]
    xp = jnp.pad(x, ((0, NP - NN), (0, 0)))
    xyzp = jnp.pad(xyz_moving, ((0, NP - NN), (0, 1)))

    w1_hm = W1.reshape(D, H, C).transpose(1, 0, 2)
    w2_hm = W2.reshape(D, H, C).transpose(1, 0, 2)
    b1_hm = b1.reshape(H, 1, C)
    b2_hm = b2.reshape(H, 1, C)

    offs = (jnp.arange(H, dtype=jnp.int32) * NP)[:, None]
    srcg = (src[None, :] + offs).reshape(-1)
    dstg = (dst[None, :] + offs).reshape(-1)

    xyzn = _normalize_xyz(xyzp)
    dif = _edge_vectors(xyzn, src, dst)
    ef1, ef2 = _edge_features(dif, jnp.asarray(_S16), _split_we(We1),
                              _split_we(We2))

    xl1 = _matmul_hm(xp, w1_hm).reshape(H * NP, C)
    num1, den1 = _gat_sc(xl1, ef1.reshape(H * EE, C), srcg, dstg, dst,
                         att1.reshape(-1))
    h1_hm = _mid(num1.reshape(H, NP, C), den1.reshape(H, NP, 1), b1_hm)
    h1 = h1_hm.transpose(1, 0, 2).reshape(NP, D)

    xl2 = _matmul_hm(h1, w2_hm).reshape(H * NP, C)
    num2, den2 = _gat_sc(xl2, ef2.reshape(H * EE, C), srcg, dstg, dst,
                         att2.reshape(-1))
    xp_hm = xp.reshape(NP, H, C).transpose(1, 0, 2)
    out_hm = _final(num2.reshape(H, NP, C), den2.reshape(H, NP, 1), b2_hm,
                    xp_hm)
    return out_hm.transpose(1, 0, 2).reshape(NP, D)[:NN]


# speedup vs baseline: 16.6292x; 1.2104x over previous
import functools

import jax
import jax.numpy as jnp
import numpy as np
from jax import lax
from jax.experimental import pallas as pl
from jax.experimental.pallas import tpu as pltpu
from jax.experimental.pallas import tpu_sc as plsc

NN = 40962
EE = 245760
H = 4
C = 32
D = 128
NP = 41472
NC = 2
NS = 16
K = 128
EW = EE // (NC * NS)
EH = EE // NS
RQ = NP // NS
RB = RQ // 3

_S16 = np.zeros((16, 12), np.float32)
for _k, _f in enumerate((1.0, 2.0, 4.0, 8.0)):
    for _c in range(3):
        _S16[_c, 3 * _k + _c] = _f


def _split_we(we):
    wv = jnp.pad(we[0:3], ((0, 13), (0, 0)))
    ws = jnp.concatenate([we[3:6], we[9:12], we[15:18], we[21:24]], axis=0)
    wc = jnp.concatenate([we[6:9], we[12:15], we[18:21], we[24:27]], axis=0)
    hm = lambda w, r: w.reshape(r, H, C).transpose(1, 0, 2)
    return hm(wv, 16), hm(ws, 12), hm(wc, 12)


def _norm_body(v_ref, o_ref):
    v = v_ref[...]
    n2 = jnp.sum(v * v, axis=1, keepdims=True)
    xn = v / jnp.sqrt(n2)
    o_ref[...] = jnp.concatenate(
        [xn, jnp.zeros((v.shape[0], 12), jnp.float32)], axis=1)


def _normalize_xyz(xyzp):
    return pl.pallas_call(
        _norm_body,
        grid=(NP // 512,),
        in_specs=[pl.BlockSpec((512, 4), lambda i: (i, 0))],
        out_specs=pl.BlockSpec((512, 16), lambda i: (i, 0)),
        out_shape=jax.ShapeDtypeStruct((NP, 16), jnp.float32),
    )(xyzp)


def _mm_body(x_ref, w_ref, o_ref):
    o_ref[0] = jnp.dot(x_ref[...], w_ref[0],
                       preferred_element_type=jnp.float32)


def _matmul_hm(xp, w_hm):
    return pl.pallas_call(
        _mm_body,
        grid=(NP // 512, H),
        in_specs=[pl.BlockSpec((512, D), lambda i, h: (i, 0)),
                  pl.BlockSpec((1, D, C), lambda i, h: (h, 0, 0))],
        out_specs=pl.BlockSpec((1, 512, C), lambda i, h: (h, i, 0)),
        out_shape=jax.ShapeDtypeStruct((H, NP, C), jnp.float32),
    )(xp, w_hm)


def _ef_body(d_ref, s_ref, wv1_ref, ws1_ref, wc1_ref, wv2_ref, ws2_ref,
             wc2_ref, o1_ref, o2_ref):
    d = d_ref[...]
    m = jnp.dot(d, s_ref[...], preferred_element_type=jnp.float32)
    sm = jnp.sin(m)
    cm = jnp.cos(m)
    for h in range(H):
        o1_ref[h] = (
            jnp.dot(d, wv1_ref[h], preferred_element_type=jnp.float32)
            + jnp.dot(sm, ws1_ref[h], preferred_element_type=jnp.float32)
            + jnp.dot(cm, wc1_ref[h], preferred_element_type=jnp.float32))
        o2_ref[h] = (
            jnp.dot(d, wv2_ref[h], preferred_element_type=jnp.float32)
            + jnp.dot(sm, ws2_ref[h], preferred_element_type=jnp.float32)
            + jnp.dot(cm, wc2_ref[h], preferred_element_type=jnp.float32))


def _edge_features(dif, s16, wsplit1, wsplit2):
    wv1, ws1, wc1 = wsplit1
    wv2, ws2, wc2 = wsplit2
    full = lambda shape: pl.BlockSpec(shape, lambda i: tuple(
        0 for _ in shape))
    return pl.pallas_call(
        _ef_body,
        grid=(EE // 1024,),
        in_specs=[pl.BlockSpec((1024, 16), lambda i: (i, 0)),
                  full((16, 12)),
                  full((H, 16, C)), full((H, 12, C)), full((H, 12, C)),
                  full((H, 16, C)), full((H, 12, C)), full((H, 12, C))],
        out_specs=[pl.BlockSpec((H, 1024, C), lambda i: (0, i, 0)),
                   pl.BlockSpec((H, 1024, C), lambda i: (0, i, 0))],
        out_shape=[jax.ShapeDtypeStruct((H, EE, C), jnp.float32),
                   jax.ShapeDtypeStruct((H, EE, C), jnp.float32)],
    )(dif, s16, wv1, ws1, wc1, wv2, ws2, wc2)


_RBC = np.zeros((H, D), np.float32)
for _h in range(H):
    _RBC[_h, _h * C:(_h + 1) * C] = 1.0


def _mid_body(n_ref, d_ref, r_ref, b_ref, o_ref):
    db = jnp.dot(d_ref[...], r_ref[...], preferred_element_type=jnp.float32)
    y = n_ref[...] / (db + 1e-16) + b_ref[...]
    o_ref[...] = jnp.where(y >= 0, y, 0.01 * y)


def _mid(num, dent, b_row):
    return pl.pallas_call(
        _mid_body,
        grid=(NP // 512,),
        in_specs=[pl.BlockSpec((512, D), lambda i: (i, 0)),
                  pl.BlockSpec((512, H), lambda i: (i, 0)),
                  pl.BlockSpec((H, D), lambda i: (0, 0)),
                  pl.BlockSpec((1, D), lambda i: (0, 0))],
        out_specs=pl.BlockSpec((512, D), lambda i: (i, 0)),
        out_shape=jax.ShapeDtypeStruct((NP, D), jnp.float32),
    )(num, dent, jnp.asarray(_RBC), b_row)


def _final_body(n_ref, d_ref, r_ref, b_ref, x_ref, o_ref):
    db = jnp.dot(d_ref[...], r_ref[...], preferred_element_type=jnp.float32)
    o_ref[...] = n_ref[...] / (db + 1e-16) + b_ref[...] + x_ref[...]


def _final(num, dent, b_row, xp):
    return pl.pallas_call(
        _final_body,
        grid=(NP // 512,),
        in_specs=[pl.BlockSpec((512, D), lambda i: (i, 0)),
                  pl.BlockSpec((512, H), lambda i: (i, 0)),
                  pl.BlockSpec((H, D), lambda i: (0, 0)),
                  pl.BlockSpec((1, D), lambda i: (0, 0)),
                  pl.BlockSpec((512, D), lambda i: (i, 0))],
        out_specs=pl.BlockSpec((512, D), lambda i: (i, 0)),
        out_shape=jax.ShapeDtypeStruct((NP, D), jnp.float32),
    )(num, dent, jnp.asarray(_RBC), b_row, xp)


_MESH = plsc.VectorSubcoreMesh(core_axis_name="c", subcore_axis_name="s")
_SC_PARAMS = pltpu.CompilerParams(use_tc_tiling_on_sc=False,
                                  needs_layout_passes=False)


@functools.partial(
    pl.kernel, mesh=_MESH,
    compiler_params=_SC_PARAMS,
    out_type=jax.ShapeDtypeStruct((EE, 16), jnp.float32),
    scratch_types=[pltpu.VMEM((K,), jnp.int32),
                   pltpu.VMEM((K,), jnp.int32),
                   pltpu.VMEM((K, 16), jnp.float32),
                   pltpu.VMEM((K, 16), jnp.float32),
                   pltpu.SemaphoreType.DMA,
                   pltpu.SemaphoreType.DMA])
def _edge_vectors(xyzn_hbm, src_hbm, dst_hbm, dif_hbm, sv, dv, xs, xd,
                  sem1, sem2):
    wid = lax.axis_index("s") * NC + lax.axis_index("c")
    base = wid * EW

    @pl.loop(0, EW // K)
    def _chunk(ci):
        e0 = base + ci * K
        pltpu.sync_copy(src_hbm.at[pl.ds(e0, K)], sv)
        pltpu.sync_copy(dst_hbm.at[pl.ds(e0, K)], dv)
        g1 = pltpu.async_copy(xyzn_hbm.at[sv], xs, sem1)
        g2 = pltpu.async_copy(xyzn_hbm.at[dv], xd, sem2)
        g1.wait()
        g2.wait()

        @pl.loop(0, K)
        def _row(i):
            xs[i, pl.ds(0, 16)] = xs[i, pl.ds(0, 16)] - xd[i, pl.ds(0, 16)]

        pltpu.sync_copy(xs, dif_hbm.at[pl.ds(e0, K)])


def _gat_sc(xl2d, ef2d, srcg, dstg, dst, att):
    @functools.partial(
        pl.kernel, mesh=_MESH,
        compiler_params=_SC_PARAMS,
        out_type=(jax.ShapeDtypeStruct((NP, D), jnp.float32),
                  jax.ShapeDtypeStruct((H * NP,), jnp.float32)),
        scratch_types=[pltpu.VMEM_SHARED((NP, C), jnp.float32),
                       pltpu.VMEM_SHARED((NP,), jnp.float32),
                       pltpu.VMEM((K,), jnp.int32),
                       pltpu.VMEM((K,), jnp.int32),
                       pltpu.VMEM((K,), jnp.int32),
                       pltpu.VMEM((K, C), jnp.float32),
                       pltpu.VMEM((K, C), jnp.float32),
                       pltpu.VMEM((K, C), jnp.float32),
                       pltpu.VMEM((K,), jnp.float32),
                       pltpu.VMEM((C,), jnp.float32),
                       pltpu.VMEM((RB, C), jnp.float32),
                       pltpu.VMEM((RB,), jnp.float32),
                       pltpu.SemaphoreType.DMA,
                       pltpu.SemaphoreType.DMA])
    def body(xl_hbm, ef_hbm, srcg_hbm, dstg_hbm, dst_hbm, att_hbm,
             num_hbm, den_hbm,
             num_acc, den_acc, sgv, dgv, dpv, xiv, xjv, efv, exb,
             attv, cbuf, dbuf, sem1, sem2):
        cc = lax.axis_index("c")
        ss = lax.axis_index("s")
        r0 = ss * RQ
        for hp in range(2):
            hh = cc * 2 + hp

            @pl.loop(0, RB)
            def _zrow(r):
                z = jnp.zeros((16,), jnp.float32)
                cbuf[r, pl.ds(0, 16)] = z
                cbuf[r, pl.ds(16, 16)] = z

            @pl.loop(0, RB // 16)
            def _zden(i):
                dbuf[pl.ds(i * 16, 16)] = jnp.zeros((16,), jnp.float32)

            for q in range(3):
                pltpu.sync_copy(cbuf, num_acc.at[pl.ds(r0 + q * RB, RB)])
                pltpu.sync_copy(dbuf, den_acc.at[pl.ds(r0 + q * RB, RB)])
            plsc.subcore_barrier()

            pltpu.sync_copy(att_hbm.at[pl.ds(hh * C, C)], attv)
            av0 = attv[pl.ds(0, 16)]
            av1 = attv[pl.ds(16, 16)]
            atts = [av0[cidx] for cidx in range(16)] \
                + [av1[cidx] for cidx in range(16)]
            lane = lax.iota(jnp.int32, 16)
            ebase = ss * EH

            @pl.loop(0, EH // K)
            def _chunk(ci):
                e0 = ebase + ci * K
                ge0 = hh * EE + e0
                pltpu.sync_copy(srcg_hbm.at[pl.ds(ge0, K)], sgv)
                pltpu.sync_copy(dstg_hbm.at[pl.ds(ge0, K)], dgv)
                pltpu.sync_copy(dst_hbm.at[pl.ds(e0, K)], dpv)
                g1 = pltpu.async_copy(xl_hbm.at[sgv], xjv, sem1)
                g2 = pltpu.async_copy(xl_hbm.at[dgv], xiv, sem2)
                pltpu.sync_copy(ef_hbm.at[pl.ds(ge0, K)], efv)
                g1.wait()
                g2.wait()

                @pl.loop(0, K)
                def _leaky(i):
                    t0 = xiv[i, pl.ds(0, 16)] + xjv[i, pl.ds(0, 16)] \
                        + efv[i, pl.ds(0, 16)]
                    t1 = xiv[i, pl.ds(16, 16)] + xjv[i, pl.ds(16, 16)] \
                        + efv[i, pl.ds(16, 16)]
                    xiv[i, pl.ds(0, 16)] = jnp.where(t0 >= 0, t0, 0.2 * t0)
                    xiv[i, pl.ds(16, 16)] = jnp.where(t1 >= 0, t1, 0.2 * t1)

                @pl.loop(0, K // 16)
                def _alpha(g):
                    e = g * 16 + lane
                    acc = jnp.zeros((16,), jnp.float32)
                    for cidx in range(C):
                        col = jnp.full((16,), cidx, jnp.int32)
                        acc = acc + plsc.load_gather(xiv, [e, col]) * atts[cidx]
                    exb[pl.ds(g * 16, 16)] = jnp.exp(acc)

                @pl.loop(0, K // 16)
                def _scale(g):
                    exv = exb[pl.ds(g * 16, 16)]
                    for l in range(16):
                        i = g * 16 + l
                        ex = exv[l]
                        xjv[i, pl.ds(0, 16)] = xjv[i, pl.ds(0, 16)] * ex
                        xjv[i, pl.ds(16, 16)] = xjv[i, pl.ds(16, 16)] * ex

                pltpu.sync_copy(xjv, num_acc.at[dpv], add=True)
                pltpu.sync_copy(exb, den_acc.at[dpv], add=True)

            plsc.subcore_barrier()
            for q in range(3):
                pltpu.sync_copy(num_acc.at[pl.ds(r0 + q * RB, RB)], cbuf)
                pltpu.sync_copy(
                    cbuf, num_hbm.at[pl.ds(r0 + q * RB, RB),
                                     pl.ds(hh * C, C)])
                pltpu.sync_copy(den_acc.at[pl.ds(r0 + q * RB, RB)], dbuf)
                pltpu.sync_copy(
                    dbuf, den_hbm.at[pl.ds(hh * NP + r0 + q * RB, RB)])
            plsc.subcore_barrier()

    return body(xl2d, ef2d, srcg, dstg, dst, att)


def kernel(x, xyz_moving, edge_index, W1, att1, We1, b1, W2, att2, We2, b2):
    src = edge_index[0]
    dst = edge_index[1]
    xp = jnp.pad(x, ((0, NP - NN), (0, 0)))
    xyzp = jnp.pad(xyz_moving, ((0, NP - NN), (0, 1)))

    w1_hm = W1.reshape(D, H, C).transpose(1, 0, 2)
    w2_hm = W2.reshape(D, H, C).transpose(1, 0, 2)
    b1_row = b1.reshape(1, D)
    b2_row = b2.reshape(1, D)

    offs = (jnp.arange(H, dtype=jnp.int32) * NP)[:, None]
    srcg = (src[None, :] + offs).reshape(-1)
    dstg = (dst[None, :] + offs).reshape(-1)

    xyzn = _normalize_xyz(xyzp)
    dif = _edge_vectors(xyzn, src, dst)
    ef1, ef2 = _edge_features(dif, jnp.asarray(_S16), _split_we(We1),
                              _split_we(We2))

    xl1 = _matmul_hm(xp, w1_hm).reshape(H * NP, C)
    num1, den1 = _gat_sc(xl1, ef1.reshape(H * EE, C), srcg, dstg, dst,
                         att1.reshape(-1))
    h1 = _mid(num1, den1.reshape(H, NP).T, b1_row)

    xl2 = _matmul_hm(h1, w2_hm).reshape(H * NP, C)
    num2, den2 = _gat_sc(xl2, ef2.reshape(H * EE, C), srcg, dstg, dst,
                         att2.reshape(-1))
    out = _final(num2, den2.reshape(H, NP).T, b2_row, xp)
    return out[:NN]


# speedup vs baseline: 21.7053x; 1.3053x over previous
import functools

import jax
import jax.numpy as jnp
import numpy as np
from jax import lax
from jax.experimental import pallas as pl
from jax.experimental.pallas import tpu as pltpu
from jax.experimental.pallas import tpu_sc as plsc

NN = 40962
EE = 245760
H = 4
C = 32
D = 128
NP = 41472
NC = 2
NS = 16
K = 128
EW = EE // (NC * NS)
EH = EE // NS
RQ = NP // NS
NQ = 6
RB = RQ // NQ

_S16 = np.zeros((16, 12), np.float32)
for _k, _f in enumerate((1.0, 2.0, 4.0, 8.0)):
    for _c in range(3):
        _S16[_c, 3 * _k + _c] = _f


def _split_we(we):
    wv = jnp.pad(we[0:3], ((0, 13), (0, 0)))
    ws = jnp.concatenate([we[3:6], we[9:12], we[15:18], we[21:24]], axis=0)
    wc = jnp.concatenate([we[6:9], we[12:15], we[18:21], we[24:27]], axis=0)
    hm = lambda w, r: w.reshape(r, H, C).transpose(1, 0, 2)
    return hm(wv, 16), hm(ws, 12), hm(wc, 12)


def _norm_body(v_ref, o_ref):
    v = v_ref[...]
    n2 = jnp.sum(v * v, axis=1, keepdims=True)
    xn = v / jnp.sqrt(n2)
    o_ref[...] = jnp.concatenate(
        [xn, jnp.zeros((v.shape[0], 12), jnp.float32)], axis=1)


def _normalize_xyz(xyzp):
    return pl.pallas_call(
        _norm_body,
        grid=(NP // 512,),
        in_specs=[pl.BlockSpec((512, 4), lambda i: (i, 0))],
        out_specs=pl.BlockSpec((512, 16), lambda i: (i, 0)),
        out_shape=jax.ShapeDtypeStruct((NP, 16), jnp.float32),
    )(xyzp)


def _mm_body(x_ref, w_ref, o_ref):
    x = x_ref[...]
    for h in range(H):
        o_ref[h] = jnp.dot(x, w_ref[h], preferred_element_type=jnp.float32)


def _matmul_hm(xp, w_hm):
    return pl.pallas_call(
        _mm_body,
        grid=(NP // 512,),
        in_specs=[pl.BlockSpec((512, D), lambda i: (i, 0)),
                  pl.BlockSpec((H, D, C), lambda i: (0, 0, 0))],
        out_specs=pl.BlockSpec((H, 512, C), lambda i: (0, i, 0)),
        out_shape=jax.ShapeDtypeStruct((H, NP, C), jnp.float32),
    )(xp, w_hm)


def _ef_body(d_ref, s_ref, wv1_ref, ws1_ref, wc1_ref, wv2_ref, ws2_ref,
             wc2_ref, *o_refs):
    d = d_ref[...]
    m = jnp.dot(d, s_ref[...], preferred_element_type=jnp.float32)
    sm = jnp.sin(m)
    cm = jnp.cos(m)
    for h in range(H):
        o_refs[h][...] = (
            jnp.dot(d, wv1_ref[h], preferred_element_type=jnp.float32)
            + jnp.dot(sm, ws1_ref[h], preferred_element_type=jnp.float32)
            + jnp.dot(cm, wc1_ref[h], preferred_element_type=jnp.float32))
        o_refs[H + h][...] = (
            jnp.dot(d, wv2_ref[h], preferred_element_type=jnp.float32)
            + jnp.dot(sm, ws2_ref[h], preferred_element_type=jnp.float32)
            + jnp.dot(cm, wc2_ref[h], preferred_element_type=jnp.float32))


def _edge_features(dif, s16, wsplit1, wsplit2):
    wv1, ws1, wc1 = wsplit1
    wv2, ws2, wc2 = wsplit2
    full = lambda shape: pl.BlockSpec(shape, lambda i: tuple(
        0 for _ in shape))
    return pl.pallas_call(
        _ef_body,
        grid=(EE // 1024,),
        in_specs=[pl.BlockSpec((1024, 16), lambda i: (i, 0)),
                  full((16, 12)),
                  full((H, 16, C)), full((H, 12, C)), full((H, 12, C)),
                  full((H, 16, C)), full((H, 12, C)), full((H, 12, C))],
        out_specs=[pl.BlockSpec((1024, C), lambda i: (i, 0))] * (2 * H),
        out_shape=[jax.ShapeDtypeStruct((EE, C), jnp.float32)] * (2 * H),
    )(dif, s16, wv1, ws1, wc1, wv2, ws2, wc2)


_RBC = np.zeros((H, D), np.float32)
for _h in range(H):
    _RBC[_h, _h * C:(_h + 1) * C] = 1.0


def _mid_body(n_ref, d_ref, r_ref, b_ref, o_ref):
    db = jnp.dot(d_ref[...], r_ref[...], preferred_element_type=jnp.float32)
    y = n_ref[...] / (db + 1e-16) + b_ref[...]
    o_ref[...] = jnp.where(y >= 0, y, 0.01 * y)


def _mid(num, dent, b_row):
    return pl.pallas_call(
        _mid_body,
        grid=(NP // 512,),
        in_specs=[pl.BlockSpec((512, D), lambda i: (i, 0)),
                  pl.BlockSpec((512, H), lambda i: (i, 0)),
                  pl.BlockSpec((H, D), lambda i: (0, 0)),
                  pl.BlockSpec((1, D), lambda i: (0, 0))],
        out_specs=pl.BlockSpec((512, D), lambda i: (i, 0)),
        out_shape=jax.ShapeDtypeStruct((NP, D), jnp.float32),
    )(num, dent, jnp.asarray(_RBC), b_row)


def _final_body(n_ref, d_ref, r_ref, b_ref, x_ref, o_ref):
    db = jnp.dot(d_ref[...], r_ref[...], preferred_element_type=jnp.float32)
    o_ref[...] = n_ref[...] / (db + 1e-16) + b_ref[...] + x_ref[...]


def _final(num, dent, b_row, xp):
    return pl.pallas_call(
        _final_body,
        grid=(NP // 512,),
        in_specs=[pl.BlockSpec((512, D), lambda i: (i, 0)),
                  pl.BlockSpec((512, H), lambda i: (i, 0)),
                  pl.BlockSpec((H, D), lambda i: (0, 0)),
                  pl.BlockSpec((1, D), lambda i: (0, 0)),
                  pl.BlockSpec((512, D), lambda i: (i, 0))],
        out_specs=pl.BlockSpec((512, D), lambda i: (i, 0)),
        out_shape=jax.ShapeDtypeStruct((NP, D), jnp.float32),
    )(num, dent, jnp.asarray(_RBC), b_row, xp)


_MESH = plsc.VectorSubcoreMesh(core_axis_name="c", subcore_axis_name="s")
_SC_PARAMS = pltpu.CompilerParams(use_tc_tiling_on_sc=False,
                                  needs_layout_passes=False)


@functools.partial(
    pl.kernel, mesh=_MESH,
    compiler_params=_SC_PARAMS,
    out_type=jax.ShapeDtypeStruct((EE, 16), jnp.float32),
    scratch_types=[pltpu.VMEM((K,), jnp.int32),
                   pltpu.VMEM((K,), jnp.int32),
                   pltpu.VMEM((K, 16), jnp.float32),
                   pltpu.VMEM((K, 16), jnp.float32),
                   pltpu.SemaphoreType.DMA,
                   pltpu.SemaphoreType.DMA])
def _edge_vectors(xyzn_hbm, src_hbm, dst_hbm, dif_hbm, sv, dv, xs, xd,
                  sem1, sem2):
    wid = lax.axis_index("s") * NC + lax.axis_index("c")
    base = wid * EW

    @pl.loop(0, EW // K)
    def _chunk(ci):
        e0 = base + ci * K
        pltpu.sync_copy(src_hbm.at[pl.ds(e0, K)], sv)
        pltpu.sync_copy(dst_hbm.at[pl.ds(e0, K)], dv)
        g1 = pltpu.async_copy(xyzn_hbm.at[sv], xs, sem1)
        g2 = pltpu.async_copy(xyzn_hbm.at[dv], xd, sem2)
        g1.wait()
        g2.wait()

        @pl.loop(0, K)
        def _row(i):
            xs[i, pl.ds(0, 16)] = xs[i, pl.ds(0, 16)] - xd[i, pl.ds(0, 16)]

        pltpu.sync_copy(xs, dif_hbm.at[pl.ds(e0, K)])


def _gat_sc(xl2d, efs, idxall, att):
    @functools.partial(
        pl.kernel, mesh=_MESH,
        compiler_params=_SC_PARAMS,
        out_type=(jax.ShapeDtypeStruct((NP, D), jnp.float32),
                  jax.ShapeDtypeStruct((H * NP,), jnp.float32)),
        scratch_types=[pltpu.VMEM_SHARED((NP, C), jnp.float32),
                       pltpu.VMEM_SHARED((NP,), jnp.float32),
                       pltpu.VMEM((3, K), jnp.int32),
                       pltpu.VMEM((3, K), jnp.int32),
                       pltpu.VMEM((K, C), jnp.float32),
                       pltpu.VMEM((K, C), jnp.float32),
                       pltpu.VMEM((K, C), jnp.float32),
                       pltpu.VMEM((K, C), jnp.float32),
                       pltpu.VMEM((K, C), jnp.float32),
                       pltpu.VMEM((K, C), jnp.float32),
                       pltpu.VMEM((K,), jnp.float32),
                       pltpu.VMEM((K,), jnp.float32),
                       pltpu.VMEM((C,), jnp.float32),
                       pltpu.VMEM((RB, C), jnp.float32),
                       pltpu.VMEM((RB,), jnp.float32)]
        + [pltpu.SemaphoreType.DMA] * 10)
    def body(xl_hbm, ef0_hbm, ef1_hbm, ef2_hbm, ef3_hbm, idx_hbm, att_hbm,
             num_hbm, den_hbm,
             num_acc, den_acc, ib0, ib1, xiv0, xiv1, xjv0, xjv1, efv0, efv1,
             exb0, exb1, attv, cbuf, dbuf,
             semi0, semi1, semj0, semj1, seme0, seme1,
             semn0, semn1, semd0, semd1):
        cc = lax.axis_index("c")
        ss = lax.axis_index("s")
        r0 = ss * RQ
        ib = [ib0, ib1]
        xiv = [xiv0, xiv1]
        xjv = [xjv0, xjv1]
        efv = [efv0, efv1]
        exb = [exb0, exb1]
        semi = [semi0, semi1]
        semj = [semj0, semj1]
        seme = [seme0, seme1]
        semn = [semn0, semn1]
        semd = [semd0, semd1]
        efh = [ef0_hbm, ef1_hbm, ef2_hbm, ef3_hbm]
        nch = EH // K
        lane = lax.iota(jnp.int32, 16)

        def head_pass(hh, ef_hbm):
            rowb = hh * (EE // K) + ss * nch

            @pl.loop(0, RB)
            def _zrow(r):
                z = jnp.zeros((16,), jnp.float32)
                cbuf[r, pl.ds(0, 16)] = z
                cbuf[r, pl.ds(16, 16)] = z

            @pl.loop(0, RB // 16)
            def _zden(i):
                dbuf[pl.ds(i * 16, 16)] = jnp.zeros((16,), jnp.float32)

            for q in range(NQ):
                pltpu.sync_copy(cbuf, num_acc.at[pl.ds(r0 + q * RB, RB)])
                pltpu.sync_copy(dbuf, den_acc.at[pl.ds(r0 + q * RB, RB)])
            plsc.subcore_barrier()

            pltpu.sync_copy(att_hbm.at[pl.ds(hh * C, C)], attv)
            av0 = attv[pl.ds(0, 16)]
            av1 = attv[pl.ds(16, 16)]
            atts = [av0[cidx] for cidx in range(16)] \
                + [av1[cidx] for cidx in range(16)]

            def issue(b, ci):
                pltpu.sync_copy(idx_hbm.at[rowb + ci], ib[b])
                pltpu.async_copy(xl_hbm.at[ib[b].at[0]], xjv[b], semj[b])
                pltpu.async_copy(xl_hbm.at[ib[b].at[1]], xiv[b], semi[b])
                pltpu.async_copy(
                    ef_hbm.at[pl.ds((ss * nch + ci) * K, K)], efv[b],
                    seme[b])

            def wait_data(b, ci):
                pltpu.make_async_copy(
                    xl_hbm.at[ib[b].at[0]], xjv[b], semj[b]).wait()
                pltpu.make_async_copy(
                    xl_hbm.at[ib[b].at[1]], xiv[b], semi[b]).wait()
                pltpu.make_async_copy(
                    ef_hbm.at[pl.ds((ss * nch + ci) * K, K)], efv[b],
                    seme[b]).wait()

            def drain_scatter(b):
                pltpu.make_async_copy(
                    xjv[b], num_acc.at[ib[b].at[2]], semn[b]).wait()
                pltpu.make_async_copy(
                    exb[b], den_acc.at[ib[b].at[2]], semd[b]).wait()

            issue(0, 0)

            @pl.loop(0, nch // 2)
            def _pair(cp):
                for b in range(2):
                    ci = cp * 2 + b
                    nb = 1 - b
                    wait_data(b, ci)

                    @pl.when(ci + 1 < nch)
                    def _prefetch():
                        if b == 0:
                            @pl.when(cp > 0)
                            def _dr():
                                drain_scatter(nb)
                        else:
                            drain_scatter(nb)
                        issue(nb, ci + 1)

                    xi, xj, ef, ex = xiv[b], xjv[b], efv[b], exb[b]

                    @pl.loop(0, K)
                    def _leaky(i):
                        t0 = xi[i, pl.ds(0, 16)] + xj[i, pl.ds(0, 16)] \
                            + ef[i, pl.ds(0, 16)]
                        t1 = xi[i, pl.ds(16, 16)] + xj[i, pl.ds(16, 16)] \
                            + ef[i, pl.ds(16, 16)]
                        xi[i, pl.ds(0, 16)] = jnp.where(t0 >= 0, t0, 0.2 * t0)
                        xi[i, pl.ds(16, 16)] = jnp.where(t1 >= 0, t1,
                                                         0.2 * t1)

                    @pl.loop(0, K // 16)
                    def _alpha(g):
                        e = g * 16 + lane
                        acc = jnp.zeros((16,), jnp.float32)
                        for cidx in range(C):
                            col = jnp.full((16,), cidx, jnp.int32)
                            acc = acc + plsc.load_gather(xi, [e, col]) \
                                * atts[cidx]
                        ex[pl.ds(g * 16, 16)] = jnp.exp(acc)

                    @pl.loop(0, K // 16)
                    def _scale(g):
                        exv = ex[pl.ds(g * 16, 16)]
                        for l in range(16):
                            i = g * 16 + l
                            exs = exv[l]
                            xj[i, pl.ds(0, 16)] = xj[i, pl.ds(0, 16)] * exs
                            xj[i, pl.ds(16, 16)] = xj[i, pl.ds(16, 16)] * exs

                    pltpu.async_copy(xj, num_acc.at[ib[b].at[2]], semn[b],
                                     add=True)
                    pltpu.async_copy(ex, den_acc.at[ib[b].at[2]], semd[b],
                                     add=True)

            drain_scatter(0)
            drain_scatter(1)
            plsc.subcore_barrier()
            for q in range(NQ):
                pltpu.sync_copy(num_acc.at[pl.ds(r0 + q * RB, RB)], cbuf)
                pltpu.sync_copy(
                    cbuf, num_hbm.at[pl.ds(r0 + q * RB, RB),
                                     pl.ds(hh * C, C)])
                pltpu.sync_copy(den_acc.at[pl.ds(r0 + q * RB, RB)], dbuf)
                pltpu.sync_copy(
                    dbuf, den_hbm.at[pl.ds(hh * NP + r0 + q * RB, RB)])
            plsc.subcore_barrier()

        for hp in range(2):
            for cv in range(2):
                @pl.when(cc == cv)
                def _run():
                    hh = cv * 2 + hp
                    head_pass(hh, efh[hh])

    return body(xl2d, efs[0], efs[1], efs[2], efs[3], idxall, att)


def kernel(x, xyz_moving, edge_index, W1, att1, We1, b1, W2, att2, We2, b2):
    src = edge_index[0]
    dst = edge_index[1]
    xp = jnp.pad(x, ((0, NP - NN), (0, 0)))
    xyzp = jnp.pad(xyz_moving, ((0, NP - NN), (0, 1)))

    w1_hm = W1.reshape(D, H, C).transpose(1, 0, 2)
    w2_hm = W2.reshape(D, H, C).transpose(1, 0, 2)
    b1_row = b1.reshape(1, D)
    b2_row = b2.reshape(1, D)

    offs = (jnp.arange(H, dtype=jnp.int32) * NP)[:, None]
    srcg3 = (src[None, :] + offs).reshape(H, EE // K, K)
    dstg3 = (dst[None, :] + offs).reshape(H, EE // K, K)
    dstt = jnp.broadcast_to(dst.reshape(1, EE // K, K), (H, EE // K, K))
    idxall = jnp.stack([srcg3, dstg3, dstt], axis=2) \
        .reshape(H * (EE // K), 3, K)

    xyzn = _normalize_xyz(xyzp)
    dif = _edge_vectors(xyzn, src, dst)
    efs = _edge_features(dif, jnp.asarray(_S16), _split_we(We1),
                         _split_we(We2))

    xl1 = _matmul_hm(xp, w1_hm).reshape(H * NP, C)
    num1, den1 = _gat_sc(xl1, efs[:H], idxall, att1.reshape(-1))
    h1 = _mid(num1, den1.reshape(H, NP).T, b1_row)

    xl2 = _matmul_hm(h1, w2_hm).reshape(H * NP, C)
    num2, den2 = _gat_sc(xl2, efs[H:], idxall, att2.reshape(-1))
    out = _final(num2, den2.reshape(H, NP).T, b2_row, xp)
    return out[:NN]


# speedup vs baseline: 31.6438x; 1.4579x over previous
import functools

import jax
import jax.numpy as jnp
import numpy as np
from jax import lax
from jax.experimental import pallas as pl
from jax.experimental.pallas import tpu as pltpu
from jax.experimental.pallas import tpu_sc as plsc

NN = 40962
EE = 245760
H = 4
C = 32
D = 128
NP = 41472
NC = 2
NS = 16
K = 128
EW = EE // (NC * NS)
EH = EE // NS
RQ = NP // NS
NQ = 6
RB = RQ // NQ

_S16 = np.zeros((16, 12), np.float32)
for _k, _f in enumerate((1.0, 2.0, 4.0, 8.0)):
    for _c in range(3):
        _S16[_c, 3 * _k + _c] = _f
_S64 = np.zeros((64, 48), np.float32)
for _j in range(4):
    _S64[_j * 16:(_j + 1) * 16, _j * 12:(_j + 1) * 12] = _S16


def _bd4(w):
    r = w.shape[0]
    z = jnp.zeros((4 * r, 4 * C), w.dtype)
    for j in range(4):
        z = z.at[j * r:(j + 1) * r, j * C:(j + 1) * C].set(w)
    return z


def _split_we(we):
    wv = jnp.pad(we[0:3], ((0, 13), (0, 0)))
    ws = jnp.concatenate([we[3:6], we[9:12], we[15:18], we[21:24]], axis=0)
    wc = jnp.concatenate([we[6:9], we[12:15], we[18:21], we[24:27]], axis=0)
    hm = lambda w, r: w.reshape(r, H, C).transpose(1, 0, 2)
    bd = lambda whm: jnp.stack([_bd4(whm[h]) for h in range(H)])
    return bd(hm(wv, 16)), bd(hm(ws, 12)), bd(hm(wc, 12))


def _norm_body(v_ref, o_ref):
    v = v_ref[...]
    n2 = jnp.sum(v * v, axis=1, keepdims=True)
    xn = v / jnp.sqrt(n2)
    o_ref[...] = jnp.concatenate(
        [xn, jnp.zeros((v.shape[0], 12), jnp.float32)], axis=1)


def _normalize_xyz(xyzp):
    return pl.pallas_call(
        _norm_body,
        grid=(NP // 512,),
        in_specs=[pl.BlockSpec((512, 4), lambda i: (i, 0))],
        out_specs=pl.BlockSpec((512, 16), lambda i: (i, 0)),
        out_shape=jax.ShapeDtypeStruct((NP, 16), jnp.float32),
    )(xyzp)


def _mm_body(x_ref, w_ref, o_ref):
    x = x_ref[...]
    for h in range(H):
        o_ref[h] = jnp.dot(x, w_ref[h], preferred_element_type=jnp.float32)


def _matmul_hm(xp, w_hm):
    return pl.pallas_call(
        _mm_body,
        grid=(NP // 512,),
        in_specs=[pl.BlockSpec((512, D), lambda i: (i, 0)),
                  pl.BlockSpec((H, D, C), lambda i: (0, 0, 0))],
        out_specs=pl.BlockSpec((H, 512, C), lambda i: (0, i, 0)),
        out_shape=jax.ShapeDtypeStruct((H, NP, C), jnp.float32),
    )(xp, w_hm)


def _ef_body(d_ref, s_ref, wv1_ref, ws1_ref, wc1_ref, wv2_ref, ws2_ref,
             wc2_ref, *o_refs):
    d = d_ref[...]
    m = jnp.dot(d, s_ref[...], preferred_element_type=jnp.float32)
    sm = jnp.sin(m)
    cm = jnp.cos(m)
    for h in range(H):
        o_refs[h][...] = (
            jnp.dot(d, wv1_ref[h], preferred_element_type=jnp.float32)
            + jnp.dot(sm, ws1_ref[h], preferred_element_type=jnp.float32)
            + jnp.dot(cm, wc1_ref[h], preferred_element_type=jnp.float32))
        o_refs[H + h][...] = (
            jnp.dot(d, wv2_ref[h], preferred_element_type=jnp.float32)
            + jnp.dot(sm, ws2_ref[h], preferred_element_type=jnp.float32)
            + jnp.dot(cm, wc2_ref[h], preferred_element_type=jnp.float32))


EE4 = EE // 4


def _edge_features(dif4, s64, wsplit1, wsplit2):
    wv1, ws1, wc1 = wsplit1
    wv2, ws2, wc2 = wsplit2
    full = lambda shape: pl.BlockSpec(shape, lambda i: tuple(
        0 for _ in shape))
    return pl.pallas_call(
        _ef_body,
        grid=(EE4 // 256,),
        in_specs=[pl.BlockSpec((256, 64), lambda i: (i, 0)),
                  full((64, 48)),
                  full((H, 64, D)), full((H, 48, D)), full((H, 48, D)),
                  full((H, 64, D)), full((H, 48, D)), full((H, 48, D))],
        out_specs=[pl.BlockSpec((256, D), lambda i: (i, 0))] * (2 * H),
        out_shape=[jax.ShapeDtypeStruct((EE4, D), jnp.float32)] * (2 * H),
    )(dif4, s64, wv1, ws1, wc1, wv2, ws2, wc2)


_RBC = np.zeros((H, D), np.float32)
for _h in range(H):
    _RBC[_h, _h * C:(_h + 1) * C] = 1.0


_DNUMS = (((0,), (0,)), ((), ()))


def _mid_body(n_ref, d_ref, r_ref, b_ref, o_ref):
    db = lax.dot_general(d_ref[...], r_ref[...], _DNUMS,
                         preferred_element_type=jnp.float32)
    y = n_ref[...] / (db + 1e-16) + b_ref[...]
    o_ref[...] = jnp.where(y >= 0, y, 0.01 * y)


def _mid(num, den, b_row):
    return pl.pallas_call(
        _mid_body,
        grid=(NP // 512,),
        in_specs=[pl.BlockSpec((512, D), lambda i: (i, 0)),
                  pl.BlockSpec((H, 512), lambda i: (0, i)),
                  pl.BlockSpec((H, D), lambda i: (0, 0)),
                  pl.BlockSpec((1, D), lambda i: (0, 0))],
        out_specs=pl.BlockSpec((512, D), lambda i: (i, 0)),
        out_shape=jax.ShapeDtypeStruct((NP, D), jnp.float32),
    )(num, den, jnp.asarray(_RBC), b_row)


def _final_body(n_ref, d_ref, r_ref, b_ref, x_ref, o_ref):
    db = lax.dot_general(d_ref[...], r_ref[...], _DNUMS,
                         preferred_element_type=jnp.float32)
    o_ref[...] = n_ref[...] / (db + 1e-16) + b_ref[...] + x_ref[...]


def _final(num, den, b_row, xp):
    return pl.pallas_call(
        _final_body,
        grid=(NP // 512,),
        in_specs=[pl.BlockSpec((512, D), lambda i: (i, 0)),
                  pl.BlockSpec((H, 512), lambda i: (0, i)),
                  pl.BlockSpec((H, D), lambda i: (0, 0)),
                  pl.BlockSpec((1, D), lambda i: (0, 0)),
                  pl.BlockSpec((512, D), lambda i: (i, 0))],
        out_specs=pl.BlockSpec((512, D), lambda i: (i, 0)),
        out_shape=jax.ShapeDtypeStruct((NP, D), jnp.float32),
    )(num, den, jnp.asarray(_RBC), b_row, xp)


_MESH = plsc.VectorSubcoreMesh(core_axis_name="c", subcore_axis_name="s")
_SC_PARAMS = pltpu.CompilerParams(use_tc_tiling_on_sc=False,
                                  needs_layout_passes=False)


@functools.partial(
    pl.kernel, mesh=_MESH,
    compiler_params=_SC_PARAMS,
    out_type=jax.ShapeDtypeStruct((EE, 16), jnp.float32),
    scratch_types=[pltpu.VMEM((K,), jnp.int32),
                   pltpu.VMEM((K,), jnp.int32),
                   pltpu.VMEM((K, 16), jnp.float32),
                   pltpu.VMEM((K, 16), jnp.float32),
                   pltpu.SemaphoreType.DMA,
                   pltpu.SemaphoreType.DMA])
def _edge_vectors(xyzn_hbm, src_hbm, dst_hbm, dif_hbm, sv, dv, xs, xd,
                  sem1, sem2):
    wid = lax.axis_index("s") * NC + lax.axis_index("c")
    base = wid * EW

    @pl.loop(0, EW // K)
    def _chunk(ci):
        e0 = base + ci * K
        pltpu.sync_copy(src_hbm.at[pl.ds(e0, K)], sv)
        pltpu.sync_copy(dst_hbm.at[pl.ds(e0, K)], dv)
        g1 = pltpu.async_copy(xyzn_hbm.at[sv], xs, sem1)
        g2 = pltpu.async_copy(xyzn_hbm.at[dv], xd, sem2)
        g1.wait()
        g2.wait()

        @pl.loop(0, K)
        def _row(i):
            xs[i, pl.ds(0, 16)] = xs[i, pl.ds(0, 16)] - xd[i, pl.ds(0, 16)]

        pltpu.sync_copy(xs, dif_hbm.at[pl.ds(e0, K)])


def _gat_sc(xl2d, efs, idxall, att):
    @functools.partial(
        pl.kernel, mesh=_MESH,
        compiler_params=_SC_PARAMS,
        out_type=(jax.ShapeDtypeStruct((NP, D), jnp.float32),
                  jax.ShapeDtypeStruct((H * NP,), jnp.float32)),
        scratch_types=[pltpu.VMEM_SHARED((NP, C), jnp.float32),
                       pltpu.VMEM_SHARED((NP,), jnp.float32),
                       pltpu.VMEM((3, K), jnp.int32),
                       pltpu.VMEM((3, K), jnp.int32),
                       pltpu.VMEM((K, C), jnp.float32),
                       pltpu.VMEM((K, C), jnp.float32),
                       pltpu.VMEM((K, C), jnp.float32),
                       pltpu.VMEM((K, C), jnp.float32),
                       pltpu.VMEM((K // 4, D), jnp.float32),
                       pltpu.VMEM((K // 4, D), jnp.float32),
                       pltpu.VMEM((K,), jnp.float32),
                       pltpu.VMEM((K,), jnp.float32),
                       pltpu.VMEM((C,), jnp.float32),
                       pltpu.VMEM((RB, C), jnp.float32),
                       pltpu.VMEM((RB,), jnp.float32)]
        + [pltpu.SemaphoreType.DMA] * 10)
    def body(xl_hbm, ef0_hbm, ef1_hbm, ef2_hbm, ef3_hbm, idx_hbm, att_hbm,
             num_hbm, den_hbm,
             num_acc, den_acc, ib0, ib1, xiv0, xiv1, xjv0, xjv1, efv0, efv1,
             exb0, exb1, attv, cbuf, dbuf,
             semi0, semi1, semj0, semj1, seme0, seme1,
             semn0, semn1, semd0, semd1):
        cc = lax.axis_index("c")
        ss = lax.axis_index("s")
        r0 = ss * RQ
        ib = [ib0, ib1]
        xiv = [xiv0, xiv1]
        xjv = [xjv0, xjv1]
        efv = [efv0, efv1]
        exb = [exb0, exb1]
        semi = [semi0, semi1]
        semj = [semj0, semj1]
        seme = [seme0, seme1]
        semn = [semn0, semn1]
        semd = [semd0, semd1]
        efh = [ef0_hbm, ef1_hbm, ef2_hbm, ef3_hbm]
        nch = EH // K
        lane = lax.iota(jnp.int32, 16)

        def head_pass(hh, ef_hbm):
            rowb = hh * (EE // K) + ss * nch

            @pl.loop(0, RB)
            def _zrow(r):
                z = jnp.zeros((16,), jnp.float32)
                cbuf[r, pl.ds(0, 16)] = z
                cbuf[r, pl.ds(16, 16)] = z

            @pl.loop(0, RB // 16)
            def _zden(i):
                dbuf[pl.ds(i * 16, 16)] = jnp.zeros((16,), jnp.float32)

            for q in range(NQ):
                pltpu.sync_copy(cbuf, num_acc.at[pl.ds(r0 + q * RB, RB)])
                pltpu.sync_copy(dbuf, den_acc.at[pl.ds(r0 + q * RB, RB)])
            plsc.subcore_barrier()

            pltpu.sync_copy(att_hbm.at[pl.ds(hh * C, C)], attv)
            av0 = attv[pl.ds(0, 16)]
            av1 = attv[pl.ds(16, 16)]
            atts = [av0[cidx] for cidx in range(16)] \
                + [av1[cidx] for cidx in range(16)]

            def issue(b, ci):
                pltpu.sync_copy(idx_hbm.at[rowb + ci], ib[b])
                pltpu.async_copy(xl_hbm.at[ib[b].at[0]], xjv[b], semj[b])
                pltpu.async_copy(xl_hbm.at[ib[b].at[1]], xiv[b], semi[b])
                pltpu.async_copy(
                    ef_hbm.at[pl.ds((ss * nch + ci) * (K // 4), K // 4)],
                    efv[b], seme[b])

            def wait_data(b, ci):
                pltpu.make_async_copy(
                    xl_hbm.at[ib[b].at[0]], xjv[b], semj[b]).wait()
                pltpu.make_async_copy(
                    xl_hbm.at[ib[b].at[1]], xiv[b], semi[b]).wait()
                pltpu.make_async_copy(
                    ef_hbm.at[pl.ds((ss * nch + ci) * (K // 4), K // 4)],
                    efv[b], seme[b]).wait()

            def drain_scatter(b):
                pltpu.make_async_copy(
                    xjv[b], num_acc.at[ib[b].at[2]], semn[b]).wait()
                pltpu.make_async_copy(
                    exb[b], den_acc.at[ib[b].at[2]], semd[b]).wait()

            issue(0, 0)

            @pl.loop(0, nch // 2)
            def _pair(cp):
                for b in range(2):
                    ci = cp * 2 + b
                    nb = 1 - b
                    wait_data(b, ci)

                    @pl.when(ci + 1 < nch)
                    def _prefetch():
                        if b == 0:
                            @pl.when(cp > 0)
                            def _dr():
                                drain_scatter(nb)
                        else:
                            drain_scatter(nb)
                        issue(nb, ci + 1)

                    xi, xj, ef, ex = xiv[b], xjv[b], efv[b], exb[b]

                    @pl.loop(0, K // 4)
                    def _leaky(q):
                        for j in range(4):
                            i = q * 4 + j
                            t0 = xi[i, pl.ds(0, 16)] + xj[i, pl.ds(0, 16)] \
                                + ef[q, pl.ds(j * C, 16)]
                            t1 = xi[i, pl.ds(16, 16)] \
                                + xj[i, pl.ds(16, 16)] \
                                + ef[q, pl.ds(j * C + 16, 16)]
                            xi[i, pl.ds(0, 16)] = jnp.where(t0 >= 0, t0,
                                                            0.2 * t0)
                            xi[i, pl.ds(16, 16)] = jnp.where(t1 >= 0, t1,
                                                             0.2 * t1)

                    @pl.loop(0, K // 16)
                    def _alpha(g):
                        e = g * 16 + lane
                        acc = jnp.zeros((16,), jnp.float32)
                        for cidx in range(C):
                            col = jnp.full((16,), cidx, jnp.int32)
                            acc = acc + plsc.load_gather(xi, [e, col]) \
                                * atts[cidx]
                        ex[pl.ds(g * 16, 16)] = jnp.exp(acc)

                    @pl.loop(0, K // 16)
                    def _scale(g):
                        exv = ex[pl.ds(g * 16, 16)]
                        for l in range(16):
                            i = g * 16 + l
                            exs = exv[l]
                            xj[i, pl.ds(0, 16)] = xj[i, pl.ds(0, 16)] * exs
                            xj[i, pl.ds(16, 16)] = xj[i, pl.ds(16, 16)] * exs

                    pltpu.async_copy(xj, num_acc.at[ib[b].at[2]], semn[b],
                                     add=True)
                    pltpu.async_copy(ex, den_acc.at[ib[b].at[2]], semd[b],
                                     add=True)

            drain_scatter(0)
            drain_scatter(1)
            plsc.subcore_barrier()
            for q in range(NQ):
                pltpu.sync_copy(num_acc.at[pl.ds(r0 + q * RB, RB)], cbuf)
                pltpu.sync_copy(
                    cbuf, num_hbm.at[pl.ds(r0 + q * RB, RB),
                                     pl.ds(hh * C, C)])
                pltpu.sync_copy(den_acc.at[pl.ds(r0 + q * RB, RB)], dbuf)
                pltpu.sync_copy(
                    dbuf, den_hbm.at[pl.ds(hh * NP + r0 + q * RB, RB)])
            plsc.subcore_barrier()

        for hp in range(2):
            for cv in range(2):
                @pl.when(cc == cv)
                def _run():
                    hh = cv * 2 + hp
                    head_pass(hh, efh[hh])

    return body(xl2d, efs[0], efs[1], efs[2], efs[3], idxall, att)


def kernel(x, xyz_moving, edge_index, W1, att1, We1, b1, W2, att2, We2, b2):
    src = edge_index[0]
    dst = edge_index[1]
    xp = jnp.pad(x, ((0, NP - NN), (0, 0)))
    xyzp = jnp.pad(xyz_moving, ((0, NP - NN), (0, 1)))

    w1_hm = W1.reshape(D, H, C).transpose(1, 0, 2)
    w2_hm = W2.reshape(D, H, C).transpose(1, 0, 2)
    b1_row = b1.reshape(1, D)
    b2_row = b2.reshape(1, D)

    offs = (jnp.arange(H, dtype=jnp.int32) * NP)[:, None]
    srcg3 = (src[None, :] + offs).reshape(H, EE // K, K)
    dstg3 = (dst[None, :] + offs).reshape(H, EE // K, K)
    dstt = jnp.broadcast_to(dst.reshape(1, EE // K, K), (H, EE // K, K))
    idxall = jnp.stack([srcg3, dstg3, dstt], axis=2) \
        .reshape(H * (EE // K), 3, K)

    xyzn = _normalize_xyz(xyzp)
    dif = _edge_vectors(xyzn, src, dst)
    efs = _edge_features(dif.reshape(EE4, 64), jnp.asarray(_S64),
                         _split_we(We1), _split_we(We2))

    xl1 = _matmul_hm(xp, w1_hm).reshape(H * NP, C)
    num1, den1 = _gat_sc(xl1, efs[:H], idxall, att1.reshape(-1))
    h1 = _mid(num1, den1.reshape(H, NP), b1_row)

    xl2 = _matmul_hm(h1, w2_hm).reshape(H * NP, C)
    num2, den2 = _gat_sc(xl2, efs[H:], idxall, att2.reshape(-1))
    out = _final(num2, den2.reshape(H, NP), b2_row, xp)
    return out[:NN]
